# baseline (speedup 1.0000x reference)
import jax
import jax.numpy as jnp
from jax import lax
from jax.experimental import pallas as pl
from jax.experimental.pallas import tpu as pltpu
from jax.experimental.pallas import tpu_sc as plsc

_BRANCH = 4096
_N = 16384
_TILES = 16
_CHUNK = _N // _TILES
_LANES = 16
_VREGS = _CHUNK // _LANES

_OFF_X = 0
_OFF_ROW = _CHUNK
_OFF_ACC = 2 * _CHUNK
_OFF_BIG = _OFF_ACC + _LANES
_OFF_OUT = _OFF_BIG + _TILES * _LANES
_BUF = _OFF_OUT + _LANES


def _sc_loss_body(idx_hbm, x_hbm, g_hbm, out_hbm,
                  idx_v, buf_v, shared, sem):
    wid = lax.axis_index("s")

    h_x = pltpu.async_copy(x_hbm.at[pl.ds(wid * _CHUNK, _CHUNK)],
                           buf_v.at[pl.ds(_OFF_X, _CHUNK)], sem)

    pltpu.sync_copy(idx_hbm, idx_v.at[pl.ds(0, 1)])
    idx0 = idx_v[...][0]

    h_r = pltpu.async_copy(g_hbm.at[idx0, pl.ds(wid * _CHUNK, _CHUNK)],
                           buf_v.at[pl.ds(_OFF_ROW, _CHUNK)], sem)
    h_r.wait()
    h_x.wait()

    acc = jnp.zeros((_LANES,), jnp.float32)
    for i in range(_VREGS):
        acc = acc + jnp.abs(buf_v[pl.ds(_OFF_X + i * _LANES, _LANES)]
                            - buf_v[pl.ds(_OFF_ROW + i * _LANES, _LANES)])

    buf_v[pl.ds(_OFF_ACC, _LANES)] = acc
    pltpu.sync_copy(buf_v.at[pl.ds(_OFF_ACC, _LANES)],
                    shared.at[pl.ds(wid * _LANES, _LANES)])
    plsc.subcore_barrier()

    @pl.when(wid == 0)
    def _():
        pltpu.sync_copy(shared, buf_v.at[pl.ds(_OFF_BIG, _TILES * _LANES)])
        total = jnp.zeros((_LANES,), jnp.float32)
        for w in range(_TILES):
            total = total + buf_v[pl.ds(_OFF_BIG + w * _LANES, _LANES)]
        lanes = lax.iota(jnp.int32, _LANES)
        dn = lax.GatherDimensionNumbers(offset_dims=(),
                                        collapsed_slice_dims=(0,),
                                        start_index_map=(0,))
        for k in (8, 4, 2, 1):
            total = total + lax.gather(
                total, (lanes ^ k)[:, None], dn, (1,),
                mode=lax.GatherScatterMode.PROMISE_IN_BOUNDS)
        buf_v[pl.ds(_OFF_OUT, _LANES)] = total * (1.0 / _N)
        pltpu.sync_copy(buf_v.at[pl.ds(_OFF_OUT, 1)], out_hbm)


@jax.jit
def _sc_loss(idx1, x, gts):
    mesh = plsc.VectorSubcoreMesh(core_axis_name="c", subcore_axis_name="s",
                                  num_cores=1)
    f = pl.kernel(
        _sc_loss_body,
        mesh=mesh,
        out_type=jax.ShapeDtypeStruct((1,), jnp.float32),
        scratch_types=[
            pltpu.VMEM((_LANES,), jnp.int32),
            pltpu.VMEM((_BUF,), jnp.float32),
            pltpu.VMEM_SHARED((_TILES * _LANES,), jnp.float32),
            pltpu.SemaphoreType.DMA,
        ],
    )
    return f(idx1, x, gts)


def kernel(x, index, gts):
    idx1 = jnp.asarray(index, jnp.int32).reshape(1)
    out = _sc_loss(idx1, x, gts)
    return out[0]


# speedup vs baseline: 1.0067x; 1.0067x over previous
import jax
import jax.numpy as jnp
from jax import lax
from jax.experimental import pallas as pl
from jax.experimental.pallas import tpu as pltpu
from jax.experimental.pallas import tpu_sc as plsc

_BRANCH = 4096
_N = 16384
_TILES = 16
_CHUNK = _N // _TILES
_LANES = 16
_VREGS = _CHUNK // _LANES

_OFF_X = 0
_OFF_ROW = _CHUNK
_OFF_ACC = 2 * _CHUNK
_OFF_BIG = _OFF_ACC + _LANES
_OFF_OUT = _OFF_BIG + _TILES * _LANES
_BUF = _OFF_OUT + _LANES


def _sc_loss_body(idx_hbm, x_hbm, g_hbm, out_hbm,
                  idx_v, buf_v, shared, sem, sem_x):
    wid = lax.axis_index("s")

    h_i = pltpu.async_copy(idx_hbm, idx_v.at[pl.ds(0, 1)], sem)
    h_x = pltpu.async_copy(x_hbm.at[pl.ds(wid * _CHUNK, _CHUNK)],
                           buf_v.at[pl.ds(_OFF_X, _CHUNK)], sem_x)
    h_i.wait()
    idx0 = idx_v[...][0]

    h_r = pltpu.async_copy(g_hbm.at[idx0, pl.ds(wid * _CHUNK, _CHUNK)],
                           buf_v.at[pl.ds(_OFF_ROW, _CHUNK)], sem)
    h_r.wait()
    h_x.wait()

    acc = jnp.zeros((_LANES,), jnp.float32)
    for i in range(_VREGS):
        acc = acc + jnp.abs(buf_v[pl.ds(_OFF_X + i * _LANES, _LANES)]
                            - buf_v[pl.ds(_OFF_ROW + i * _LANES, _LANES)])

    @pl.when(wid != 0)
    def _():
        buf_v[pl.ds(_OFF_ACC, _LANES)] = acc
        pltpu.sync_copy(buf_v.at[pl.ds(_OFF_ACC, _LANES)],
                        shared.at[pl.ds(wid * _LANES, _LANES)])
    plsc.subcore_barrier()

    @pl.when(wid == 0)
    def _():
        pltpu.sync_copy(
            shared.at[pl.ds(_LANES, (_TILES - 1) * _LANES)],
            buf_v.at[pl.ds(_OFF_BIG, (_TILES - 1) * _LANES)])
        total = acc
        for w in range(_TILES - 1):
            total = total + buf_v[pl.ds(_OFF_BIG + w * _LANES, _LANES)]
        lanes = lax.iota(jnp.int32, _LANES)
        dn = lax.GatherDimensionNumbers(offset_dims=(),
                                        collapsed_slice_dims=(0,),
                                        start_index_map=(0,))
        for k in (8, 4, 2, 1):
            total = total + lax.gather(
                total, (lanes ^ k)[:, None], dn, (1,),
                mode=lax.GatherScatterMode.PROMISE_IN_BOUNDS)
        buf_v[pl.ds(_OFF_OUT, _LANES)] = total * (1.0 / _N)
        pltpu.sync_copy(buf_v.at[pl.ds(_OFF_OUT, 1)], out_hbm)


@jax.jit
def _sc_loss(idx1, x, gts):
    mesh = plsc.VectorSubcoreMesh(core_axis_name="c", subcore_axis_name="s",
                                  num_cores=1)
    f = pl.kernel(
        _sc_loss_body,
        mesh=mesh,
        out_type=jax.ShapeDtypeStruct((1,), jnp.float32),
        scratch_types=[
            pltpu.VMEM((_LANES,), jnp.int32),
            pltpu.VMEM((_BUF,), jnp.float32),
            pltpu.VMEM_SHARED((_TILES * _LANES,), jnp.float32),
            pltpu.SemaphoreType.DMA,
            pltpu.SemaphoreType.DMA,
        ],
    )
    return f(idx1, x, gts)


def kernel(x, index, gts):
    idx1 = jnp.asarray(index, jnp.int32).reshape(1)
    out = _sc_loss(idx1, x, gts)
    return out[0]


# speedup vs baseline: 1.0073x; 1.0007x over previous
import jax
import jax.numpy as jnp
from jax import lax
from jax.experimental import pallas as pl
from jax.experimental.pallas import tpu as pltpu
from jax.experimental.pallas import tpu_sc as plsc

_BRANCH = 4096
_N = 16384
_TILES = 16
_CHUNK = _N // _TILES
_LANES = 16
_VREGS = _CHUNK // _LANES

_OFF_X = 0
_OFF_ROW = _CHUNK
_OFF_ACC = 2 * _CHUNK
_OFF_BIG = _OFF_ACC + _LANES
_OFF_OUT = _OFF_BIG + _TILES * _LANES
_BUF = _OFF_OUT + _LANES


def _sc_loss_body(idx_hbm, x_hbm, g_hbm, out_hbm,
                  idx_v, buf_v, shared, sem, sem_x):
    wid = lax.axis_index("s")

    h_i = pltpu.async_copy(idx_hbm, idx_v.at[pl.ds(0, 1)], sem)
    h_x = pltpu.async_copy(x_hbm.at[pl.ds(wid * _CHUNK, _CHUNK)],
                           buf_v.at[pl.ds(_OFF_X, _CHUNK)], sem_x)
    h_i.wait()
    idx0 = idx_v[...][0]

    h_r = pltpu.async_copy(g_hbm.at[idx0, pl.ds(wid * _CHUNK, _CHUNK)],
                           buf_v.at[pl.ds(_OFF_ROW, _CHUNK)], sem)
    h_r.wait()
    h_x.wait()

    @plsc.parallel_loop(0, _CHUNK, step=_LANES, unroll=8,
                        carry=jnp.zeros((_LANES,), jnp.float32))
    def acc(i, a):
        return a + jnp.abs(buf_v[pl.ds(_OFF_X + i, _LANES)]
                           - buf_v[pl.ds(_OFF_ROW + i, _LANES)])

    @pl.when(wid != 0)
    def _():
        buf_v[pl.ds(_OFF_ACC, _LANES)] = acc
        pltpu.sync_copy(buf_v.at[pl.ds(_OFF_ACC, _LANES)],
                        shared.at[pl.ds(wid * _LANES, _LANES)])
    plsc.subcore_barrier()

    @pl.when(wid == 0)
    def _():
        pltpu.sync_copy(
            shared.at[pl.ds(_LANES, (_TILES - 1) * _LANES)],
            buf_v.at[pl.ds(_OFF_BIG, (_TILES - 1) * _LANES)])
        total = acc
        for w in range(_TILES - 1):
            total = total + buf_v[pl.ds(_OFF_BIG + w * _LANES, _LANES)]
        lanes = lax.iota(jnp.int32, _LANES)
        dn = lax.GatherDimensionNumbers(offset_dims=(),
                                        collapsed_slice_dims=(0,),
                                        start_index_map=(0,))
        for k in (8, 4, 2, 1):
            total = total + lax.gather(
                total, (lanes ^ k)[:, None], dn, (1,),
                mode=lax.GatherScatterMode.PROMISE_IN_BOUNDS)
        buf_v[pl.ds(_OFF_OUT, _LANES)] = total * (1.0 / _N)
        pltpu.sync_copy(buf_v.at[pl.ds(_OFF_OUT, 1)], out_hbm)


@jax.jit
def _sc_loss(idx1, x, gts):
    mesh = plsc.VectorSubcoreMesh(core_axis_name="c", subcore_axis_name="s",
                                  num_cores=1)
    f = pl.kernel(
        _sc_loss_body,
        mesh=mesh,
        out_type=jax.ShapeDtypeStruct((1,), jnp.float32),
        scratch_types=[
            pltpu.VMEM((_LANES,), jnp.int32),
            pltpu.VMEM((_BUF,), jnp.float32),
            pltpu.VMEM_SHARED((_TILES * _LANES,), jnp.float32),
            pltpu.SemaphoreType.DMA,
            pltpu.SemaphoreType.DMA,
        ],
    )
    return f(idx1, x, gts)


def kernel(x, index, gts):
    idx1 = jnp.asarray(index, jnp.int32).reshape(1)
    out = _sc_loss(idx1, x, gts)
    return out[0]


# speedup vs baseline: 1.0134x; 1.0061x over previous
import jax
import jax.numpy as jnp
from jax import lax
from jax.experimental import pallas as pl
from jax.experimental.pallas import tpu as pltpu
from jax.experimental.pallas import tpu_sc as plsc

_BRANCH = 4096
_N = 16384
_TILES = 16
_CHUNK = _N // _TILES
_LANES = 16
_VREGS = _CHUNK // _LANES

_OFF_X = 0
_OFF_ROW = _CHUNK
_OFF_ACC = 2 * _CHUNK
_OFF_BIG = _OFF_ACC + _LANES
_OFF_OUT = _OFF_BIG + _TILES * _LANES
_BUF = _OFF_OUT + _LANES


def _sc_loss_body(idx_hbm, x_hbm, g_hbm, out_hbm,
                  idx_v, buf_v, shared, sem, sem_x):
    wid = lax.axis_index("s")

    h_i = pltpu.async_copy(idx_hbm, idx_v.at[pl.ds(0, 1)], sem)
    h_x = pltpu.async_copy(x_hbm.at[pl.ds(wid * _CHUNK, _CHUNK)],
                           buf_v.at[pl.ds(_OFF_X, _CHUNK)], sem_x)
    h_i.wait()
    idx0 = idx_v[...][0]

    h_r = pltpu.async_copy(g_hbm.at[idx0, pl.ds(wid * _CHUNK, _CHUNK)],
                           buf_v.at[pl.ds(_OFF_ROW, _CHUNK)], sem)
    h_r.wait()
    h_x.wait()

    @plsc.parallel_loop(0, _CHUNK, step=_LANES, unroll=16,
                        carry=jnp.zeros((_LANES,), jnp.float32))
    def acc(i, a):
        return a + jnp.abs(buf_v[pl.ds(_OFF_X + i, _LANES)]
                           - buf_v[pl.ds(_OFF_ROW + i, _LANES)])

    @pl.when(wid != 0)
    def _():
        buf_v[pl.ds(_OFF_ACC, _LANES)] = acc
        pltpu.sync_copy(buf_v.at[pl.ds(_OFF_ACC, _LANES)],
                        shared.at[pl.ds(wid * _LANES, _LANES)])
    plsc.subcore_barrier()

    @pl.when(wid == 0)
    def _():
        pltpu.sync_copy(
            shared.at[pl.ds(_LANES, (_TILES - 1) * _LANES)],
            buf_v.at[pl.ds(_OFF_BIG, (_TILES - 1) * _LANES)])
        total = acc
        for w in range(_TILES - 1):
            total = total + buf_v[pl.ds(_OFF_BIG + w * _LANES, _LANES)]
        lanes = lax.iota(jnp.int32, _LANES)
        dn = lax.GatherDimensionNumbers(offset_dims=(),
                                        collapsed_slice_dims=(0,),
                                        start_index_map=(0,))
        for k in (8, 4, 2, 1):
            total = total + lax.gather(
                total, (lanes ^ k)[:, None], dn, (1,),
                mode=lax.GatherScatterMode.PROMISE_IN_BOUNDS)
        buf_v[pl.ds(_OFF_OUT, _LANES)] = total * (1.0 / _N)
        pltpu.sync_copy(buf_v.at[pl.ds(_OFF_OUT, 1)], out_hbm)


@jax.jit
def _sc_loss(idx1, x, gts):
    mesh = plsc.VectorSubcoreMesh(core_axis_name="c", subcore_axis_name="s",
                                  num_cores=1)
    f = pl.kernel(
        _sc_loss_body,
        mesh=mesh,
        out_type=jax.ShapeDtypeStruct((1,), jnp.float32),
        scratch_types=[
            pltpu.VMEM((_LANES,), jnp.int32),
            pltpu.VMEM((_BUF,), jnp.float32),
            pltpu.VMEM_SHARED((_TILES * _LANES,), jnp.float32),
            pltpu.SemaphoreType.DMA,
            pltpu.SemaphoreType.DMA,
        ],
    )
    return f(idx1, x, gts)


def kernel(x, index, gts):
    idx1 = jnp.asarray(index, jnp.int32).reshape(1)
    out = _sc_loss(idx1, x, gts)
    return out[0]
